# baseline (speedup 1.0000x reference)
import dataclasses
import functools

import jax
import jax.numpy as jnp
from jax import lax
from jax.experimental import pallas as pl
from jax.experimental.pallas import tpu as pltpu
from jax.experimental.pallas import tpu_sc as plsc

NC = 2
NS = 16
NW = NC * NS

B = 8192
D = 4096
BPW = B // NW
NBUF = 8
NFB = 4


def _sc_embed(ids_flat, wte3):
    mesh = plsc.VectorSubcoreMesh(core_axis_name="c", subcore_axis_name="s")
    cp = pltpu.CompilerParams()
    if "needs_layout_passes" in pltpu.CompilerParams.__dataclass_fields__:
        cp = dataclasses.replace(cp, needs_layout_passes=False)

    @functools.partial(
        pl.kernel,
        compiler_params=cp,
        out_type=jax.ShapeDtypeStruct((B * D,), jnp.float32),
        mesh=mesh,
        scratch_types=(
            [pltpu.VMEM((BPW,), jnp.int32)]
            + [pltpu.VMEM((2, D), jnp.bfloat16)] * NBUF
            + [pltpu.VMEM((D,), jnp.float32)] * NFB
            + [pltpu.SemaphoreType.DMA] * (NBUF + NFB)
        ),
    )
    def k(ids_hbm, wte_hbm, out_hbm, idx_v, *bufs):
        slabs = bufs[:NBUF]
        fouts = bufs[NBUF:NBUF + NFB]
        gsem = bufs[NBUF + NFB:2 * NBUF + NFB]
        wsem = bufs[2 * NBUF + NFB:]
        wid = lax.axis_index("s") * NC + lax.axis_index("c")
        base = wid * BPW
        pltpu.sync_copy(ids_hbm.at[pl.ds(base, BPW)], idx_v)

        lanes = lax.iota(jnp.int32, 16)
        himask = jnp.uint32(0xFFFF0000)

        def row_of(u):
            tv = idx_v[pl.ds((u >> 4) * 16, 16)]
            return jnp.sum(jnp.where(lanes == (u & 15), tv, 0))

        for b in range(NBUF):
            pltpu.make_async_copy(
                wte_hbm.at[row_of(b) >> 1], slabs[b], gsem[b]
            ).start()

        @pl.loop(0, BPW // NBUF)
        def _chunk(cc):
            for b in range(NBUF):
                t = cc * NBUF + b
                pltpu.make_async_copy(
                    wte_hbm.at[0], slabs[b], gsem[b]
                ).wait()

                row = row_of(t)
                r = row & 1
                sh = jnp.where((r & 1) == 0, 16, 0).astype(jnp.uint32)
                fb = b % NFB

                @pl.when(t >= NFB)
                def _():
                    pltpu.make_async_copy(
                        fouts[fb], out_hbm.at[pl.ds(0, D)], wsem[fb]
                    ).wait()

                @plsc.parallel_loop(0, D // 16, unroll=8)
                def _col(j, r=r, sh=sh, fb=fb, b=b):
                    x = slabs[b][r, pl.ds(j * 16, 32)]
                    w = plsc.bitcast(x, jnp.uint32)
                    y = plsc.bitcast((w << sh) & himask, jnp.float32)
                    plsc.store_scatter(fouts[fb], [j * 16 + lanes], y)

                pltpu.make_async_copy(
                    fouts[fb], out_hbm.at[pl.ds((base + t) * D, D)],
                    wsem[fb],
                ).start()

                @pl.when(t + NBUF < BPW)
                def _():
                    pltpu.make_async_copy(
                        wte_hbm.at[row_of(t + NBUF) >> 1], slabs[b],
                        gsem[b],
                    ).start()

        for fb in range(NFB):
            pltpu.make_async_copy(
                fouts[fb], out_hbm.at[pl.ds(0, D)], wsem[fb]
            ).wait()

    return k(ids_flat, wte3)


def kernel(input_ids, wte):
    ids_flat = input_ids.reshape(-1).astype(jnp.int32)
    wte3 = wte.reshape(wte.shape[0] // 2, 2, D)
    out = _sc_embed(ids_flat, wte3)
    return out.reshape(input_ids.shape[0], input_ids.shape[1], D)


# speedup vs baseline: 6.8691x; 6.8691x over previous
import dataclasses
import functools

import jax
import jax.numpy as jnp
from jax import lax
from jax.experimental import pallas as pl
from jax.experimental.pallas import tpu as pltpu
from jax.experimental.pallas import tpu_sc as plsc

NC = 2
NS = 16
NW = NC * NS

B = 8192
D = 4096
BPW = B // NW
NBUF = 4
NFB = 4


def _sc_embed(ids_flat, wte3):
    mesh = plsc.VectorSubcoreMesh(core_axis_name="c", subcore_axis_name="s")
    cp = pltpu.CompilerParams()
    if "needs_layout_passes" in pltpu.CompilerParams.__dataclass_fields__:
        cp = dataclasses.replace(cp, needs_layout_passes=False)

    @functools.partial(
        pl.kernel,
        compiler_params=cp,
        out_type=jax.ShapeDtypeStruct((B * D,), jnp.float32),
        mesh=mesh,
        scratch_types=(
            [pltpu.VMEM((BPW,), jnp.int32)]
            + [pltpu.VMEM((8, D), jnp.bfloat16)] * NBUF
            + [pltpu.VMEM((D,), jnp.float32)] * NFB
            + [pltpu.SemaphoreType.DMA] * (NBUF + NFB)
        ),
    )
    def k(ids_hbm, wte_hbm, out_hbm, idx_v, *bufs):
        slabs = bufs[:NBUF]
        fouts = bufs[NBUF:NBUF + NFB]
        gsem = bufs[NBUF + NFB:2 * NBUF + NFB]
        wsem = bufs[2 * NBUF + NFB:]
        wid = lax.axis_index("s") * NC + lax.axis_index("c")
        base = wid * BPW
        pltpu.sync_copy(ids_hbm.at[pl.ds(base, BPW)], idx_v)

        lanes = lax.iota(jnp.int32, 16)
        himask = jnp.uint32(0xFFFF0000)

        def row_of(u):
            tv = idx_v[pl.ds((u >> 4) * 16, 16)]
            return jnp.sum(jnp.where(lanes == (u & 15), tv, 0))

        for b in range(NBUF):
            pltpu.make_async_copy(
                wte_hbm.at[row_of(b) >> 3], slabs[b], gsem[b]
            ).start()

        @pl.loop(0, BPW // NBUF)
        def _chunk(cc):
            for b in range(NBUF):
                t = cc * NBUF + b
                pltpu.make_async_copy(
                    wte_hbm.at[0], slabs[b], gsem[b]
                ).wait()

                row = row_of(t)
                r = row & 7
                sh = jnp.where((r & 1) == 0, 16, 0).astype(jnp.uint32)
                fb = b % NFB

                @pl.when(t >= NFB)
                def _():
                    pltpu.make_async_copy(
                        fouts[fb], out_hbm.at[pl.ds(0, D)], wsem[fb]
                    ).wait()

                @plsc.parallel_loop(0, D // 16, unroll=8)
                def _col(j, r=r, sh=sh, fb=fb, b=b):
                    x = slabs[b][r, pl.ds(j * 16, 32)]
                    w = plsc.bitcast(x, jnp.uint32)
                    y = plsc.bitcast((w << sh) & himask, jnp.float32)
                    plsc.store_scatter(fouts[fb], [j * 16 + lanes], y)

                pltpu.make_async_copy(
                    fouts[fb], out_hbm.at[pl.ds((base + t) * D, D)],
                    wsem[fb],
                ).start()

                @pl.when(t + NBUF < BPW)
                def _():
                    pltpu.make_async_copy(
                        wte_hbm.at[row_of(t + NBUF) >> 3], slabs[b],
                        gsem[b],
                    ).start()

        for fb in range(NFB):
            pltpu.make_async_copy(
                fouts[fb], out_hbm.at[pl.ds(0, D)], wsem[fb]
            ).wait()

    return k(ids_flat, wte3)


def kernel(input_ids, wte):
    ids_flat = input_ids.reshape(-1).astype(jnp.int32)
    wte3 = wte.reshape(wte.shape[0] // 8, 8, D)
    out = _sc_embed(ids_flat, wte3)
    return out.reshape(input_ids.shape[0], input_ids.shape[1], D)
